# baseline (speedup 1.0000x reference)
import functools

import jax
import jax.numpy as jnp
from jax import lax
from jax.experimental import pallas as pl
from jax.experimental.pallas import tpu as pltpu
from jax.experimental.pallas import tpu_sc as plsc

T = 4096
H = 768
F = 3072
E = 64
C = 80
S = E * C
NB = E + 1
NF = 1
FC = F // NF

NC = 2
NS = 16
NW = NC * NS
DISP_PER_W = S // NW
DISP_CHUNK = 80
COMB_PER_W = T // NW


RCH = 512
RNC = T // RCH


def _router_body(x_ref, rw_ref, stt_ref, gslot_ref, dst_ref,
                 carry_ref, hi_ref, lo_ref, g_ref):
    i = pl.program_id(0)
    xv = x_ref[...]
    logits = jnp.dot(xv, rw_ref[...], preferred_element_type=jnp.float32)

    lmax = jnp.max(logits, axis=1, keepdims=True)
    iota_e = lax.broadcasted_iota(jnp.int32, (RCH, E), 1)
    is_max = logits >= lmax
    eidx = jnp.min(jnp.where(is_max, iota_e, E), axis=1, keepdims=True)
    gate = 1.0 / jnp.sum(jnp.exp(logits - lmax), axis=1, keepdims=True)

    oh = (iota_e == eidx).astype(jnp.float32)

    @pl.when(i == 0)
    def _():
        carry_ref[...] = jnp.zeros((1, E), jnp.float32)
        hi_ref[...] = jnp.zeros((E, C), jnp.float32)
        lo_ref[...] = jnp.zeros((E, C), jnp.float32)
        g_ref[...] = jnp.zeros((E, C), jnp.float32)

    r_i = lax.broadcasted_iota(jnp.int32, (RCH, RCH), 0)
    c_i = lax.broadcasted_iota(jnp.int32, (RCH, RCH), 1)
    lt = (r_i >= c_i).astype(jnp.float32)
    cum = jnp.dot(lt, oh, preferred_element_type=jnp.float32) + carry_ref[...]
    carry_ref[...] = cum[RCH - 1:RCH]

    p = (jnp.sum(oh * cum, axis=1, keepdims=True) - 1.0).astype(jnp.int32)
    keep = p < C

    iota_c = lax.broadcasted_iota(jnp.int32, (RCH, C), 1)
    pm = (p == iota_c).astype(jnp.float32)
    tok1 = i * RCH + lax.broadcasted_iota(jnp.int32, (RCH, 1), 0) + 1
    tok_hi = (tok1 >> 8).astype(jnp.float32)
    tok_lo = (tok1 & 255).astype(jnp.float32)

    dn = (((0,), (0,)), ((), ()))
    hi_ref[...] += lax.dot_general(oh, pm * tok_hi, dn,
                                   preferred_element_type=jnp.float32)
    lo_ref[...] += lax.dot_general(oh, pm * tok_lo, dn,
                                   preferred_element_type=jnp.float32)
    g_ref[...] += lax.dot_general(oh, pm * gate, dn,
                                  precision=lax.Precision.HIGHEST,
                                  preferred_element_type=jnp.float32)

    dst_ref[...] = jnp.where(keep, eidx * C + p, S)

    @pl.when(i == RNC - 1)
    def _():
        stt_f = hi_ref[...] * 256.0 + lo_ref[...] - 1.0
        slot_iota = jnp.bitwise_and(
            lax.broadcasted_iota(jnp.int32, (E, C), 0) * C
            + lax.broadcasted_iota(jnp.int32, (E, C), 1), T - 1)
        stt_ref[...] = jnp.where(stt_f < 0.0, slot_iota,
                                 stt_f.astype(jnp.int32))
        gslot_ref[0:E, 0] = g_ref[...]
        gslot_ref[E:NB, 0] = jnp.zeros((1, C), jnp.float32)


def _sc_dispatch_body(x_hbm, stt_hbm, xb_hbm, idx0, idx1, rows0, rows1,
                      gsem0, gsem1, ssem0, ssem1):
    wid = lax.axis_index("s") * NC + lax.axis_index("c")
    base = wid * DISP_PER_W
    pltpu.sync_copy(stt_hbm.at[2 * wid], idx0)
    pltpu.sync_copy(stt_hbm.at[2 * wid + 1], idx1)
    g0 = pltpu.async_copy(x_hbm.at[idx0], rows0, gsem0)
    g1 = pltpu.async_copy(x_hbm.at[idx1], rows1, gsem1)
    g0.wait()
    s0 = pltpu.async_copy(rows0, xb_hbm.at[pl.ds(base, DISP_CHUNK)], ssem0)
    g1.wait()
    s1 = pltpu.async_copy(rows1, xb_hbm.at[pl.ds(base + DISP_CHUNK, DISP_CHUNK)],
                          ssem1)
    s0.wait()
    s1.wait()


def _sc_combine_body(yb_hbm, dst_hbm, out_hbm, idx_v, rows_v, sem):
    wid = lax.axis_index("s") * NC + lax.axis_index("c")
    base = wid * COMB_PER_W
    pltpu.sync_copy(dst_hbm.at[pl.ds(base, COMB_PER_W)], idx_v)
    pltpu.async_copy(yb_hbm.at[idx_v], rows_v, sem).wait()
    pltpu.sync_copy(rows_v, out_hbm.at[pl.ds(base, COMB_PER_W)])


def _ffn_body(xb_ref, w1_ref, w2_ref, g_ref, out_ref, acc_ref):
    f = pl.program_id(1)
    h = jnp.dot(xb_ref[0], w1_ref[0], preferred_element_type=jnp.float32)
    h = h * (1.0 / (1.0 + jnp.exp(-h)))
    y = jnp.dot(h, w2_ref[0], preferred_element_type=jnp.float32)

    @pl.when(f == 0)
    def _():
        acc_ref[...] = y

    @pl.when(f != 0)
    def _():
        acc_ref[...] = acc_ref[...] + y

    @pl.when(f == NF - 1)
    def _():
        out_ref[0] = acc_ref[...] * jnp.reshape(g_ref[0, 0], (C, 1))


_router_call = pl.pallas_call(
    _router_body,
    grid=(RNC,),
    in_specs=[
        pl.BlockSpec((RCH, H), lambda i: (i, 0)),
        pl.BlockSpec((H, E), lambda i: (0, 0)),
    ],
    out_specs=(
        pl.BlockSpec((E, C), lambda i: (0, 0)),
        pl.BlockSpec((NB, 1, C), lambda i: (0, 0, 0)),
        pl.BlockSpec((RCH, 1), lambda i: (i, 0)),
    ),
    out_shape=(
        jax.ShapeDtypeStruct((E, C), jnp.int32),
        jax.ShapeDtypeStruct((NB, 1, C), jnp.float32),
        jax.ShapeDtypeStruct((T, 1), jnp.int32),
    ),
    scratch_shapes=[
        pltpu.VMEM((1, E), jnp.float32),
        pltpu.VMEM((E, C), jnp.float32),
        pltpu.VMEM((E, C), jnp.float32),
        pltpu.VMEM((E, C), jnp.float32),
    ],
)

_ffn_call = pl.pallas_call(
    _ffn_body,
    grid=(NB, NF),
    in_specs=[
        pl.BlockSpec((1, C, H), lambda e, f: (jnp.minimum(e, E - 1), 0, 0)),
        pl.BlockSpec((1, H, FC), lambda e, f: (jnp.minimum(e, E - 1), 0, f)),
        pl.BlockSpec((1, FC, H), lambda e, f: (jnp.minimum(e, E - 1), f, 0)),
        pl.BlockSpec((1, 1, C), lambda e, f: (e, 0, 0)),
    ],
    out_specs=pl.BlockSpec((1, C, H), lambda e, f: (e, 0, 0)),
    out_shape=jax.ShapeDtypeStruct((NB, C, H), jnp.float32),
    scratch_shapes=[pltpu.VMEM((C, H), jnp.float32)],
)

_sc_mesh = plsc.VectorSubcoreMesh(
    core_axis_name="c", subcore_axis_name="s", num_cores=NC, num_subcores=NS)

_dispatch_call = pl.kernel(
    _sc_dispatch_body,
    out_type=jax.ShapeDtypeStruct((S, H), jnp.float32),
    mesh=_sc_mesh,
    scratch_types=[
        pltpu.VMEM((DISP_CHUNK,), jnp.int32),
        pltpu.VMEM((DISP_CHUNK,), jnp.int32),
        pltpu.VMEM((DISP_CHUNK, H), jnp.float32),
        pltpu.VMEM((DISP_CHUNK, H), jnp.float32),
        pltpu.SemaphoreType.DMA,
        pltpu.SemaphoreType.DMA,
        pltpu.SemaphoreType.DMA,
        pltpu.SemaphoreType.DMA,
    ],
)

_combine_call = pl.kernel(
    _sc_combine_body,
    out_type=jax.ShapeDtypeStruct((T, H), jnp.float32),
    mesh=_sc_mesh,
    scratch_types=[
        pltpu.VMEM((COMB_PER_W,), jnp.int32),
        pltpu.VMEM((COMB_PER_W, H), jnp.float32),
        pltpu.SemaphoreType.DMA,
    ],
)


def kernel(x, router_w, w1, w2):
    stt, gpad, dst = _router_call(x, router_w)

    xb = _dispatch_call(x, stt)
    yb = _ffn_call(xb.reshape(E, C, H), w1, w2, gpad)
    out = _combine_call(yb.reshape(NB * C, H), dst.reshape(T))
    return out


# speedup vs baseline: 1.0070x; 1.0070x over previous
import functools

import jax
import jax.numpy as jnp
from jax import lax
from jax.experimental import pallas as pl
from jax.experimental.pallas import tpu as pltpu
from jax.experimental.pallas import tpu_sc as plsc

T = 4096
H = 768
F = 3072
E = 64
C = 80
S = E * C
NB = E + 1
NF = 1
FC = F // NF

NC = 2
NS = 16
NW = NC * NS
DISP_PER_W = S // NW
DISP_CHUNK = 80
COMB_PER_W = T // NW


def _router_body(x_ref, rw_ref, stt_ref, gslot_ref, dst_ref):
    xv = x_ref[...]
    logits = jnp.dot(xv, rw_ref[...], preferred_element_type=jnp.float32)

    lmax = jnp.max(logits, axis=1, keepdims=True)
    iota_e = lax.broadcasted_iota(jnp.int32, (T, E), 1)
    is_max = logits >= lmax
    eidx = jnp.min(jnp.where(is_max, iota_e, E), axis=1, keepdims=True)
    gate = 1.0 / jnp.sum(jnp.exp(logits - lmax), axis=1, keepdims=True)

    oh = (iota_e == eidx).astype(jnp.float32)

    CH = 512
    r_i = lax.broadcasted_iota(jnp.int32, (CH, CH), 0)
    c_i = lax.broadcasted_iota(jnp.int32, (CH, CH), 1)
    lt = (r_i >= c_i).astype(jnp.float32)
    carry = jnp.zeros((1, E), jnp.float32)
    cums = []
    for i in range(T // CH):
        blk = oh[i * CH:(i + 1) * CH]
        cb = jnp.dot(lt, blk, preferred_element_type=jnp.float32) + carry
        cums.append(cb)
        carry = cb[CH - 1:CH]
    cum = jnp.concatenate(cums, axis=0)

    p = (jnp.sum(oh * cum, axis=1, keepdims=True) - 1.0).astype(jnp.int32)
    keep = p < C

    iota_c = lax.broadcasted_iota(jnp.int32, (T, C), 1)
    pm = (p == iota_c).astype(jnp.float32)
    tok1 = lax.broadcasted_iota(jnp.int32, (T, 1), 0) + 1
    tok_hi = (tok1 >> 8).astype(jnp.float32)
    tok_lo = (tok1 & 255).astype(jnp.float32)

    dn = (((0,), (0,)), ((), ()))
    stt_hi = lax.dot_general(oh, pm * tok_hi, dn,
                             preferred_element_type=jnp.float32)
    stt_lo = lax.dot_general(oh, pm * tok_lo, dn,
                             preferred_element_type=jnp.float32)
    stt_f = stt_hi * 256.0 + stt_lo - 1.0
    slot_iota = jnp.bitwise_and(
        lax.broadcasted_iota(jnp.int32, (E, C), 0) * C
        + lax.broadcasted_iota(jnp.int32, (E, C), 1), T - 1)
    stt_i = jnp.where(stt_f < 0.0, slot_iota, stt_f.astype(jnp.int32))
    g_f = lax.dot_general(oh, pm * gate, dn, precision=lax.Precision.HIGHEST,
                          preferred_element_type=jnp.float32)

    dst = jnp.where(keep, eidx * C + p, S)
    stt_ref[...] = stt_i
    gslot_ref[0:E, 0] = g_f
    gslot_ref[E:NB, 0] = jnp.zeros((1, C), jnp.float32)
    dst_ref[...] = dst


def _sc_dispatch_body(x_hbm, stt_hbm, xb_hbm, idx0, idx1, rows0, rows1,
                      gsem0, gsem1, ssem0, ssem1):
    wid = lax.axis_index("s") * NC + lax.axis_index("c")
    base = wid * DISP_PER_W
    pltpu.sync_copy(stt_hbm.at[2 * wid], idx0)
    pltpu.sync_copy(stt_hbm.at[2 * wid + 1], idx1)
    g0 = pltpu.async_copy(x_hbm.at[idx0], rows0, gsem0)
    g1 = pltpu.async_copy(x_hbm.at[idx1], rows1, gsem1)
    g0.wait()
    s0 = pltpu.async_copy(rows0, xb_hbm.at[pl.ds(base, DISP_CHUNK)], ssem0)
    g1.wait()
    s1 = pltpu.async_copy(rows1, xb_hbm.at[pl.ds(base + DISP_CHUNK, DISP_CHUNK)],
                          ssem1)
    s0.wait()
    s1.wait()


def _sc_combine_body(yb_hbm, dst_hbm, out_hbm, idx_v, rows_v, sem):
    wid = lax.axis_index("s") * NC + lax.axis_index("c")
    base = wid * COMB_PER_W
    pltpu.sync_copy(dst_hbm.at[pl.ds(base, COMB_PER_W)], idx_v)
    pltpu.async_copy(yb_hbm.at[idx_v], rows_v, sem).wait()
    pltpu.sync_copy(rows_v, out_hbm.at[pl.ds(base, COMB_PER_W)])


def _ffn_body(xb_ref, w1_ref, w2_ref, g_ref, out_ref, acc_ref):
    f = pl.program_id(1)
    h = jnp.dot(xb_ref[0], w1_ref[0], preferred_element_type=jnp.float32)
    h = h * (1.0 / (1.0 + jnp.exp(-h)))
    y = jnp.dot(h, w2_ref[0], preferred_element_type=jnp.float32)

    @pl.when(f == 0)
    def _():
        acc_ref[...] = y

    @pl.when(f != 0)
    def _():
        acc_ref[...] = acc_ref[...] + y

    @pl.when(f == NF - 1)
    def _():
        out_ref[0] = acc_ref[...] * jnp.reshape(g_ref[0, 0], (C, 1))


_router_call = pl.pallas_call(
    _router_body,
    out_shape=(
        jax.ShapeDtypeStruct((E, C), jnp.int32),
        jax.ShapeDtypeStruct((NB, 1, C), jnp.float32),
        jax.ShapeDtypeStruct((T, 1), jnp.int32),
    ),
)

_ffn_call = pl.pallas_call(
    _ffn_body,
    grid=(NB, NF),
    in_specs=[
        pl.BlockSpec((1, C, H), lambda e, f: (jnp.minimum(e, E - 1), 0, 0)),
        pl.BlockSpec((1, H, FC), lambda e, f: (jnp.minimum(e, E - 1), 0, f)),
        pl.BlockSpec((1, FC, H), lambda e, f: (jnp.minimum(e, E - 1), f, 0)),
        pl.BlockSpec((1, 1, C), lambda e, f: (e, 0, 0)),
    ],
    out_specs=pl.BlockSpec((1, C, H), lambda e, f: (e, 0, 0)),
    out_shape=jax.ShapeDtypeStruct((NB, C, H), jnp.float32),
    scratch_shapes=[pltpu.VMEM((C, H), jnp.float32)],
)

_sc_mesh = plsc.VectorSubcoreMesh(
    core_axis_name="c", subcore_axis_name="s", num_cores=NC, num_subcores=NS)

_dispatch_call = pl.kernel(
    _sc_dispatch_body,
    out_type=jax.ShapeDtypeStruct((S, H), jnp.float32),
    mesh=_sc_mesh,
    scratch_types=[
        pltpu.VMEM((DISP_CHUNK,), jnp.int32),
        pltpu.VMEM((DISP_CHUNK,), jnp.int32),
        pltpu.VMEM((DISP_CHUNK, H), jnp.float32),
        pltpu.VMEM((DISP_CHUNK, H), jnp.float32),
        pltpu.SemaphoreType.DMA,
        pltpu.SemaphoreType.DMA,
        pltpu.SemaphoreType.DMA,
        pltpu.SemaphoreType.DMA,
    ],
)

_combine_call = pl.kernel(
    _sc_combine_body,
    out_type=jax.ShapeDtypeStruct((T, H), jnp.float32),
    mesh=_sc_mesh,
    scratch_types=[
        pltpu.VMEM((COMB_PER_W,), jnp.int32),
        pltpu.VMEM((COMB_PER_W, H), jnp.float32),
        pltpu.SemaphoreType.DMA,
    ],
)


def kernel(x, router_w, w1, w2):
    stt, gpad, dst = _router_call(x, router_w)

    xb = _dispatch_call(x, stt)
    yb = _ffn_call(xb.reshape(E, C, H), w1, w2, gpad)
    out = _combine_call(yb.reshape(NB * C, H), dst.reshape(T))
    return out
